# baseline (speedup 1.0000x reference)
import functools

import jax
import jax.numpy as jnp
from jax import lax
from jax.experimental import pallas as pl
from jax.experimental.pallas import tpu as pltpu
from jax.experimental.pallas import tpu_sc as plsc

_PC_START = (-54.0, -54.0)
_VOXEL_SIZE = (0.075, 0.075)
_OUT_STRIDE = 8

_LANES = 16


@functools.cache
def _make_sc_kernel(B, C, H, W, NPTS):
    info = plsc.get_sparse_core_info()
    NC, NS = info.num_cores, info.num_subcores
    NW = NC * NS
    wpb = NW // B
    cpw = C // wpb
    assert B * wpb == NW and wpb * cpw == C and NPTS % _LANES == 0
    nchunk = NPTS // _LANES
    UNROLL = 4
    assert nchunk % UNROLL == 0

    sx = jnp.float32(_VOXEL_SIZE[0] * _OUT_STRIDE)
    sy = jnp.float32(_VOXEL_SIZE[1] * _OUT_STRIDE)
    ox = jnp.float32(_PC_START[0])
    oy = jnp.float32(_PC_START[1])

    mesh = plsc.VectorSubcoreMesh(core_axis_name="c", subcore_axis_name="s")

    @functools.partial(
        pl.kernel,
        out_type=jax.ShapeDtypeStruct((B, C, NPTS), jnp.float32),
        mesh=mesh,
        compiler_params=pltpu.CompilerParams(use_tc_tiling_on_sc=False,
                                             needs_layout_passes=False),
        scratch_types=[
            pltpu.VMEM((H, W), jnp.float32),
            pltpu.VMEM((H, W), jnp.float32),
            pltpu.VMEM((NPTS,), jnp.float32),
            pltpu.VMEM((NPTS,), jnp.float32),
            pltpu.VMEM((NPTS,), jnp.int32),
            pltpu.VMEM((NPTS,), jnp.int32),
            pltpu.VMEM((NPTS,), jnp.float32),
            pltpu.VMEM((NPTS,), jnp.float32),
            pltpu.SemaphoreType.DMA,
            pltpu.SemaphoreType.DMA,
            pltpu.SemaphoreType.DMA,
            pltpu.SemaphoreType.DMA,
        ],
    )
    def bev_kernel(bev_hbm, x_hbm, y_hbm, out_hbm,
                   plane0, plane1, xb, yb, x0b, y0b, ob0, ob1,
                   psem0, psem1, osem0, osem1):
        wid = lax.axis_index("s") * NC + lax.axis_index("c")
        b = wid // wpb
        c0 = (wid % wpb) * cpw

        pltpu.sync_copy(x_hbm.at[b], xb)
        pltpu.sync_copy(y_hbm.at[b], yb)

        def coord_body(i, _):
            s = pl.ds(i * _LANES, _LANES)
            x = (xb[s] - ox) / sx
            y = (yb[s] - oy) / sy
            xi = x.astype(jnp.int32)
            yi = y.astype(jnp.int32)
            xi = jnp.where(xi.astype(jnp.float32) > x, xi - 1, xi)
            yi = jnp.where(yi.astype(jnp.float32) > y, yi - 1, yi)
            xb[s] = x
            yb[s] = y
            x0b[s] = jnp.clip(xi, 0, W - 1)
            y0b[s] = jnp.clip(yi, 0, H - 1)
            return 0

        lax.fori_loop(0, nchunk, coord_body, 0, unroll=2)

        planes = (plane0, plane1)
        psems = (psem0, psem1)
        obufs = (ob0, ob1)
        osems = (osem0, osem1)

        pcopies = [pltpu.async_copy(bev_hbm.at[b, c0], planes[0], psems[0]),
                   None]
        ocopies = [None, None]

        for ci in range(cpw):
            cur = ci % 2
            nxt = 1 - cur
            if ci + 1 < cpw:
                pcopies[nxt] = pltpu.async_copy(
                    bev_hbm.at[b, c0 + ci + 1], planes[nxt], psems[nxt])
            pcopies[cur].wait()
            if ocopies[cur] is not None:
                ocopies[cur].wait()
            plane = planes[cur]
            ob = obufs[cur]

            def chunk_body(i, _, plane=plane, ob=ob):
                for u in range(UNROLL):
                    s = pl.ds(i * (UNROLL * _LANES) + u * _LANES, _LANES)
                    x = xb[s]
                    y = yb[s]
                    x0 = x0b[s]
                    y0 = y0b[s]
                    x1 = jnp.minimum(x0 + 1, W - 1)
                    y1 = jnp.minimum(y0 + 1, H - 1)
                    ia = plsc.load_gather(plane, [y0, x0])
                    ib = plsc.load_gather(plane, [y1, x0])
                    ic = plsc.load_gather(plane, [y0, x1])
                    id_ = plsc.load_gather(plane, [y1, x1])
                    wx0 = x - x0.astype(jnp.float32)
                    wx1 = x1.astype(jnp.float32) - x
                    wy0 = y - y0.astype(jnp.float32)
                    wy1 = y1.astype(jnp.float32) - y
                    ob[s] = wx1 * (ia * wy1 + ib * wy0) + wx0 * (ic * wy1 + id_ * wy0)
                return 0

            lax.fori_loop(0, nchunk // UNROLL, chunk_body, 0)
            ocopies[cur] = pltpu.async_copy(ob, out_hbm.at[b, c0 + ci],
                                            osems[cur])
        for oc in ocopies:
            if oc is not None:
                oc.wait()

    return bev_kernel


def kernel(bev_feature, batch_centers, num_point):
    if isinstance(num_point, tuple):
        num_point = num_point[0] * num_point[1]
    B, C, H, W = bev_feature.shape
    _, N, P, _ = batch_centers.shape
    NPTS = N * P
    x = batch_centers[..., 0].reshape(B, NPTS)
    y = batch_centers[..., 1].reshape(B, NPTS)
    out_t = _make_sc_kernel(B, C, H, W, NPTS)(bev_feature, x, y)
    return jnp.transpose(out_t.reshape(B, C, N, P), (0, 2, 3, 1)).reshape(
        B, N, P * C)


# speedup vs baseline: 6.5469x; 6.5469x over previous
import functools

import jax
import jax.numpy as jnp
from jax import lax
from jax.experimental import pallas as pl
from jax.experimental.pallas import tpu as pltpu
from jax.experimental.pallas import tpu_sc as plsc

_PC_START = (-54.0, -54.0)
_VOXEL_SIZE = (0.075, 0.075)
_OUT_STRIDE = 8

_LANES = 16
_RL = 128


@functools.cache
def _make_sc_kernel(B, C, H, W, N, P):
    info = plsc.get_sparse_core_info()
    NC, NS = info.num_cores, info.num_subcores
    NW = NC * NS
    NPTS = N * P
    CT = C // _RL
    TN = N // 8
    TPC = P * C // _RL
    ppw = B * NPTS // NW
    nchunk = ppw // _LANES
    wpb = NW // B
    assert CT * _RL == C and TN * 8 == N and wpb * B == NW
    assert ppw * NW == B * NPTS and nchunk % 2 == 0 and P == 5

    sx = jnp.float32(_VOXEL_SIZE[0] * _OUT_STRIDE)
    sy = jnp.float32(_VOXEL_SIZE[1] * _OUT_STRIDE)
    ox = jnp.float32(_PC_START[0])
    oy = jnp.float32(_PC_START[1])

    mesh = plsc.VectorSubcoreMesh(core_axis_name="c", subcore_axis_name="s")

    @functools.partial(
        pl.kernel,
        out_type=jax.ShapeDtypeStruct((B * TN * TPC * 8, _RL), jnp.float32),
        mesh=mesh,
        compiler_params=pltpu.CompilerParams(use_tc_tiling_on_sc=False,
                                             needs_layout_passes=False),
        scratch_types=[
            pltpu.VMEM((ppw,), jnp.float32),
            pltpu.VMEM((ppw,), jnp.float32),
            pltpu.VMEM((ppw,), jnp.float32),
            pltpu.VMEM((ppw,), jnp.float32),
            pltpu.VMEM((ppw,), jnp.float32),
            pltpu.VMEM((ppw,), jnp.float32),
            pltpu.VMEM((nchunk, 8 * _LANES), jnp.int32),
            pltpu.VMEM((nchunk, 2 * _LANES), jnp.int32),
            pltpu.VMEM((8 * _LANES, _RL), jnp.float32),
            pltpu.VMEM((8 * _LANES, _RL), jnp.float32),
            pltpu.VMEM((2 * _LANES, _RL), jnp.float32),
            pltpu.VMEM((2 * _LANES, _RL), jnp.float32),
            pltpu.SemaphoreType.DMA,
            pltpu.SemaphoreType.DMA,
            pltpu.SemaphoreType.DMA,
            pltpu.SemaphoreType.DMA,
        ],
    )
    def bev_kernel(rows_hbm, x_hbm, y_hbm, out_hbm,
                   xb, yb, wab, wbb, wcb, wdb, gidx, oidx,
                   gb0, gb1, ob0, ob1, gs0, gs1, os0, os1):
        wid = lax.axis_index("s") * NC + lax.axis_index("c")
        b = wid // wpb
        j0 = (wid % wpb) * ppw

        pltpu.sync_copy(x_hbm.at[b, pl.ds(j0, ppw)], xb)
        pltpu.sync_copy(y_hbm.at[b, pl.ds(j0, ppw)], yb)

        lane = lax.iota(jnp.int32, _LANES)

        def stage_a(ci, _):
            s = pl.ds(ci * _LANES, _LANES)
            x = (xb[s] - ox) / sx
            y = (yb[s] - oy) / sy
            xi = x.astype(jnp.int32)
            yi = y.astype(jnp.int32)
            xi = jnp.where(xi.astype(jnp.float32) > x, xi - 1, xi)
            yi = jnp.where(yi.astype(jnp.float32) > y, yi - 1, yi)
            x0 = jnp.clip(xi, 0, W - 1)
            y0 = jnp.clip(yi, 0, H - 1)
            x1 = jnp.minimum(x0 + 1, W - 1)
            y1 = jnp.minimum(y0 + 1, H - 1)
            wx0 = x - x0.astype(jnp.float32)
            wx1 = x1.astype(jnp.float32) - x
            wy0 = y - y0.astype(jnp.float32)
            wy1 = y1.astype(jnp.float32) - y
            wab[s] = wx1 * wy1
            wbb[s] = wx1 * wy0
            wcb[s] = wx0 * wy1
            wdb[s] = wx0 * wy0
            ra = (y0 * W + x0) * (CT * B) + b
            rb = (y1 * W + x0) * (CT * B) + b
            rc = (y0 * W + x1) * (CT * B) + b
            rd = (y1 * W + x1) * (CT * B) + b
            gidx[ci, pl.ds(0 * _LANES, _LANES)] = ra
            gidx[ci, pl.ds(1 * _LANES, _LANES)] = ra + B
            gidx[ci, pl.ds(2 * _LANES, _LANES)] = rb
            gidx[ci, pl.ds(3 * _LANES, _LANES)] = rb + B
            gidx[ci, pl.ds(4 * _LANES, _LANES)] = rc
            gidx[ci, pl.ds(5 * _LANES, _LANES)] = rc + B
            gidx[ci, pl.ds(6 * _LANES, _LANES)] = rd
            gidx[ci, pl.ds(7 * _LANES, _LANES)] = rd + B
            jg = j0 + ci * _LANES + lane
            n = lax.shift_right_logical(jg * 6554, 15)
            p = jg - n * 5
            o0 = ((b * TN + lax.shift_right_logical(n, 3)) * TPC
                  + p * CT) * 8 + (n & 7)
            oidx[ci, pl.ds(0, _LANES)] = o0
            oidx[ci, pl.ds(_LANES, _LANES)] = o0 + 8
            return 0

        lax.fori_loop(0, nchunk, stage_a, 0)

        def combine(ci, gbuf, obuf):
            s = pl.ds(ci * _LANES, _LANES)
            wa = wab[s]
            wb = wbb[s]
            wc = wcb[s]
            wd = wdb[s]

            def point(i, _):
                bi = jnp.full((_LANES,), i, jnp.int32)
                wai = wa.at[bi].get(mode="promise_in_bounds")
                wbi = wb.at[bi].get(mode="promise_in_bounds")
                wci = wc.at[bi].get(mode="promise_in_bounds")
                wdi = wd.at[bi].get(mode="promise_in_bounds")
                for t in range(CT):
                    r = t * _LANES + i
                    for v in range(_RL // _LANES):
                        cs = pl.ds(v * _LANES, _LANES)
                        acc = (gbuf[0 * CT * _LANES + r, cs] * wai
                               + gbuf[1 * CT * _LANES + r, cs] * wbi
                               + gbuf[2 * CT * _LANES + r, cs] * wci
                               + gbuf[3 * CT * _LANES + r, cs] * wdi)
                        obuf[r, cs] = acc
                return 0

            lax.fori_loop(0, _LANES, point, 0)

        last = nchunk - 1
        pltpu.async_copy(rows_hbm.at[gidx.at[0]], gb0, gs0)
        pltpu.async_copy(rows_hbm.at[gidx.at[1]], gb1, gs1)

        def pair(hi, _):
            for par, gbuf, gsem, obuf, osem in (
                    (0, gb0, gs0, ob0, os0), (1, gb1, gs1, ob1, os1)):
                ci = hi * 2 + par
                pltpu.make_async_copy(rows_hbm.at[gidx.at[ci]], gbuf,
                                      gsem).wait()

                @pl.when(hi > 0)
                def _wait_prev_scatter(obuf=obuf, osem=osem, ci=ci):
                    pltpu.make_async_copy(obuf, out_hbm.at[oidx.at[ci - 2]],
                                          osem).wait()

                combine(ci, gbuf, obuf)
                pltpu.async_copy(obuf, out_hbm.at[oidx.at[ci]], osem)
                nxt = jnp.minimum(ci + 2, last)
                pltpu.async_copy(rows_hbm.at[gidx.at[nxt]], gbuf, gsem)
            return 0

        lax.fori_loop(0, nchunk // 2, pair, 0)

        pltpu.make_async_copy(rows_hbm.at[gidx.at[last]], gb0, gs0).wait()
        pltpu.make_async_copy(rows_hbm.at[gidx.at[last]], gb1, gs1).wait()
        pltpu.make_async_copy(ob0, out_hbm.at[oidx.at[last - 1]], os0).wait()
        pltpu.make_async_copy(ob1, out_hbm.at[oidx.at[last]], os1).wait()

    return bev_kernel


def kernel(bev_feature, batch_centers, num_point):
    if isinstance(num_point, tuple):
        num_point = num_point[0] * num_point[1]
    B, C, H, W = bev_feature.shape
    _, N, P, _ = batch_centers.shape
    NPTS = N * P
    CT = C // _RL
    TN = N // 8
    TPC = P * C // _RL
    bev_rows = jnp.transpose(
        bev_feature.reshape(B, CT, _RL, H, W), (3, 4, 1, 0, 2)
    ).reshape(H * W * CT * B, _RL)
    x = batch_centers[..., 0].reshape(B, NPTS)
    y = batch_centers[..., 1].reshape(B, NPTS)
    out_rows = _make_sc_kernel(B, C, H, W, N, P)(bev_rows, x, y)
    return jnp.transpose(
        out_rows.reshape(B, TN, TPC, 8, _RL), (0, 1, 3, 2, 4)
    ).reshape(B, N, P * C)


# speedup vs baseline: 10.7168x; 1.6369x over previous
import functools

import jax
import jax.numpy as jnp
from jax import lax
from jax.experimental import pallas as pl
from jax.experimental.pallas import tpu as pltpu
from jax.experimental.pallas import tpu_sc as plsc

_PC_START = (-54.0, -54.0)
_VOXEL_SIZE = (0.075, 0.075)
_OUT_STRIDE = 8

_LANES = 16
_RL = 128


@functools.cache
def _make_sc_kernel(B, C, H, W, N, P):
    info = plsc.get_sparse_core_info()
    NC, NS = info.num_cores, info.num_subcores
    NW = NC * NS
    NPTS = N * P
    CT = C // _RL
    TN = N // 8
    TPC = P * C // _RL
    ppw = B * NPTS // NW
    nchunk = ppw // _LANES
    wpb = NW // B
    npw = N // wpb
    assert CT * _RL == C and TN * 8 == N and wpb * B == NW
    assert ppw * NW == B * NPTS and nchunk % 2 == 0
    assert npw % (2 * _LANES) == 0 and nchunk == P * (npw // _LANES)

    sx = jnp.float32(_VOXEL_SIZE[0] * _OUT_STRIDE)
    sy = jnp.float32(_VOXEL_SIZE[1] * _OUT_STRIDE)
    ox = jnp.float32(_PC_START[0])
    oy = jnp.float32(_PC_START[1])

    mesh = plsc.VectorSubcoreMesh(core_axis_name="c", subcore_axis_name="s")

    @functools.partial(
        pl.kernel,
        out_type=jax.ShapeDtypeStruct((B * TN * TPC * 8, _RL), jnp.float32),
        mesh=mesh,
        compiler_params=pltpu.CompilerParams(use_tc_tiling_on_sc=False,
                                             needs_layout_passes=False),
        scratch_types=[
            pltpu.VMEM((P, npw), jnp.float32),
            pltpu.VMEM((P, npw), jnp.float32),
            pltpu.VMEM((ppw,), jnp.float32),
            pltpu.VMEM((ppw,), jnp.float32),
            pltpu.VMEM((ppw,), jnp.float32),
            pltpu.VMEM((ppw,), jnp.float32),
            pltpu.VMEM((nchunk, 8 * _LANES), jnp.int32),
            pltpu.VMEM((nchunk, 2 * _LANES), jnp.int32),
            pltpu.VMEM((8 * _LANES, _RL), jnp.float32),
            pltpu.VMEM((8 * _LANES, _RL), jnp.float32),
            pltpu.VMEM((2 * _LANES, _RL), jnp.float32),
            pltpu.VMEM((2 * _LANES, _RL), jnp.float32),
            pltpu.SemaphoreType.DMA,
            pltpu.SemaphoreType.DMA,
            pltpu.SemaphoreType.DMA,
            pltpu.SemaphoreType.DMA,
        ],
    )
    def bev_kernel(rows_hbm, bc_hbm, out_hbm,
                   xb, yb, wab, wbb, wcb, wdb, gidx, oidx,
                   gb0, gb1, ob0, ob1, gs0, gs1, os0, os1):
        wid = lax.axis_index("s") * NC + lax.axis_index("c")
        b = wid // wpb
        n0 = (wid % wpb) * npw

        pltpu.sync_copy(bc_hbm.at[b, :, 0, pl.ds(n0, npw)], xb)
        pltpu.sync_copy(bc_hbm.at[b, :, 1, pl.ds(n0, npw)], yb)

        lane = lax.iota(jnp.int32, _LANES)

        nc_per_p = npw // _LANES

        def stage_a(ci, _):
            p = ci // nc_per_p
            nc = ci - p * nc_per_p
            sn = pl.ds(nc * _LANES, _LANES)
            x = (xb[p, sn] - ox) / sx
            y = (yb[p, sn] - oy) / sy
            xi = x.astype(jnp.int32)
            yi = y.astype(jnp.int32)
            xi = jnp.where(xi.astype(jnp.float32) > x, xi - 1, xi)
            yi = jnp.where(yi.astype(jnp.float32) > y, yi - 1, yi)
            x0 = jnp.clip(xi, 0, W - 1)
            y0 = jnp.clip(yi, 0, H - 1)
            x1 = jnp.minimum(x0 + 1, W - 1)
            y1 = jnp.minimum(y0 + 1, H - 1)
            wx0 = x - x0.astype(jnp.float32)
            wx1 = x1.astype(jnp.float32) - x
            wy0 = y - y0.astype(jnp.float32)
            wy1 = y1.astype(jnp.float32) - y
            s = pl.ds(ci * _LANES, _LANES)
            wab[s] = wx1 * wy1
            wbb[s] = wx1 * wy0
            wcb[s] = wx0 * wy1
            wdb[s] = wx0 * wy0
            ra = (y0 * W + x0) * (CT * B) + b
            rb = (y1 * W + x0) * (CT * B) + b
            rc = (y0 * W + x1) * (CT * B) + b
            rd = (y1 * W + x1) * (CT * B) + b
            gidx[ci, pl.ds(0 * _LANES, _LANES)] = ra
            gidx[ci, pl.ds(1 * _LANES, _LANES)] = ra + B
            gidx[ci, pl.ds(2 * _LANES, _LANES)] = rb
            gidx[ci, pl.ds(3 * _LANES, _LANES)] = rb + B
            gidx[ci, pl.ds(4 * _LANES, _LANES)] = rc
            gidx[ci, pl.ds(5 * _LANES, _LANES)] = rc + B
            gidx[ci, pl.ds(6 * _LANES, _LANES)] = rd
            gidx[ci, pl.ds(7 * _LANES, _LANES)] = rd + B
            n = n0 + nc * _LANES + lane
            o0 = ((b * TN + lax.shift_right_logical(n, 3)) * TPC
                  + p * CT) * 8 + (n & 7)
            oidx[ci, pl.ds(0, _LANES)] = o0
            oidx[ci, pl.ds(_LANES, _LANES)] = o0 + 8
            return 0

        lax.fori_loop(0, nchunk, stage_a, 0)

        def combine(ci, gbuf, obuf):
            s = pl.ds(ci * _LANES, _LANES)
            wa = wab[s]
            wb = wbb[s]
            wc = wcb[s]
            wd = wdb[s]

            @plsc.parallel_loop(0, _LANES, step=1, unroll=2)
            def point(i):
                bi = jnp.full((_LANES,), i, jnp.int32)
                wai = wa.at[bi].get(mode="promise_in_bounds")
                wbi = wb.at[bi].get(mode="promise_in_bounds")
                wci = wc.at[bi].get(mode="promise_in_bounds")
                wdi = wd.at[bi].get(mode="promise_in_bounds")
                for t in range(CT):
                    r = t * _LANES + i
                    for v in range(_RL // _LANES):
                        cs = pl.ds(v * _LANES, _LANES)
                        acc = (gbuf[0 * CT * _LANES + r, cs] * wai
                               + gbuf[1 * CT * _LANES + r, cs] * wbi
                               + gbuf[2 * CT * _LANES + r, cs] * wci
                               + gbuf[3 * CT * _LANES + r, cs] * wdi)
                        obuf[r, cs] = acc

        last = nchunk - 1
        pltpu.async_copy(rows_hbm.at[gidx.at[0]], gb0, gs0)
        pltpu.async_copy(rows_hbm.at[gidx.at[1]], gb1, gs1)

        def pair(hi, _):
            for par, gbuf, gsem, obuf, osem in (
                    (0, gb0, gs0, ob0, os0), (1, gb1, gs1, ob1, os1)):
                ci = hi * 2 + par
                pltpu.make_async_copy(rows_hbm.at[gidx.at[ci]], gbuf,
                                      gsem).wait()

                @pl.when(hi > 0)
                def _wait_prev_scatter(obuf=obuf, osem=osem, ci=ci):
                    pltpu.make_async_copy(obuf, out_hbm.at[oidx.at[ci - 2]],
                                          osem).wait()

                combine(ci, gbuf, obuf)
                pltpu.async_copy(obuf, out_hbm.at[oidx.at[ci]], osem)
                nxt = jnp.minimum(ci + 2, last)
                pltpu.async_copy(rows_hbm.at[gidx.at[nxt]], gbuf, gsem)
            return 0

        lax.fori_loop(0, nchunk // 2, pair, 0)

        pltpu.make_async_copy(rows_hbm.at[gidx.at[last]], gb0, gs0).wait()
        pltpu.make_async_copy(rows_hbm.at[gidx.at[last]], gb1, gs1).wait()
        pltpu.make_async_copy(ob0, out_hbm.at[oidx.at[last - 1]], os0).wait()
        pltpu.make_async_copy(ob1, out_hbm.at[oidx.at[last]], os1).wait()

    return bev_kernel


def kernel(bev_feature, batch_centers, num_point):
    if isinstance(num_point, tuple):
        num_point = num_point[0] * num_point[1]
    B, C, H, W = bev_feature.shape
    _, N, P, _ = batch_centers.shape
    CT = C // _RL
    TN = N // 8
    TPC = P * C // _RL
    bev_rows = jnp.transpose(
        bev_feature.reshape(B, CT, _RL, H, W), (3, 4, 1, 0, 2)
    ).reshape(H * W * CT * B, _RL)
    bc_view = jnp.transpose(batch_centers, (0, 2, 3, 1))
    out_rows = _make_sc_kernel(B, C, H, W, N, P)(bev_rows, bc_view)
    return jnp.transpose(
        out_rows.reshape(B, TN, TPC, 8, _RL), (0, 1, 3, 2, 4)
    ).reshape(B, N, P * C)


# speedup vs baseline: 10.7781x; 1.0057x over previous
import functools

import jax
import jax.numpy as jnp
from jax import lax
from jax.experimental import pallas as pl
from jax.experimental.pallas import tpu as pltpu
from jax.experimental.pallas import tpu_sc as plsc

_PC_START = (-54.0, -54.0)
_VOXEL_SIZE = (0.075, 0.075)
_OUT_STRIDE = 8

_LANES = 16
_RL = 128


@functools.cache
def _make_sc_kernel(B, C, H, W, N, P):
    info = plsc.get_sparse_core_info()
    NC, NS = info.num_cores, info.num_subcores
    NW = NC * NS
    NPTS = N * P
    CT = C // _RL
    TN = N // 8
    TPC = P * C // _RL
    ppw = B * NPTS // NW
    nchunk = ppw // _LANES
    wpb = NW // B
    npw = N // wpb
    assert CT * _RL == C and TN * 8 == N and wpb * B == NW
    assert ppw * NW == B * NPTS and nchunk % 2 == 0
    assert npw == _RL and nchunk == P * (npw // _LANES)

    sx = float(_VOXEL_SIZE[0] * _OUT_STRIDE)
    sy = float(_VOXEL_SIZE[1] * _OUT_STRIDE)
    ox = float(_PC_START[0])
    oy = float(_PC_START[1])

    mesh = plsc.VectorSubcoreMesh(core_axis_name="c", subcore_axis_name="s")

    @functools.partial(
        pl.kernel,
        out_type=jax.ShapeDtypeStruct((B * TN * TPC * 8, _RL), jnp.float32),
        mesh=mesh,
        compiler_params=pltpu.CompilerParams(use_tc_tiling_on_sc=False,
                                             needs_layout_passes=False),
        scratch_types=[
            pltpu.VMEM((P, npw), jnp.float32),
            pltpu.VMEM((P, npw), jnp.float32),
            pltpu.VMEM((ppw,), jnp.float32),
            pltpu.VMEM((ppw,), jnp.float32),
            pltpu.VMEM((ppw,), jnp.float32),
            pltpu.VMEM((ppw,), jnp.float32),
            pltpu.VMEM((nchunk, 8 * _LANES), jnp.int32),
            pltpu.VMEM((nchunk, 2 * _LANES), jnp.int32),
            pltpu.VMEM((8 * _LANES, _RL), jnp.float32),
            pltpu.VMEM((8 * _LANES, _RL), jnp.float32),
            pltpu.VMEM((2 * _LANES, _RL), jnp.float32),
            pltpu.VMEM((2 * _LANES, _RL), jnp.float32),
            pltpu.SemaphoreType.DMA,
            pltpu.SemaphoreType.DMA,
            pltpu.SemaphoreType.DMA,
            pltpu.SemaphoreType.DMA,
        ],
    )
    def bev_kernel(rows_hbm, bc_hbm, out_hbm,
                   xb, yb, wab, wbb, wcb, wdb, gidx, oidx,
                   gb0, gb1, ob0, ob1, gs0, gs1, os0, os1):
        wid = lax.axis_index("s") * NC + lax.axis_index("c")
        b = wid // wpb
        n0 = (wid % wpb) * npw

        nb = wid % wpb
        pltpu.sync_copy(bc_hbm.at[b, :, nb, 0, :], xb)
        pltpu.sync_copy(bc_hbm.at[b, :, nb, 1, :], yb)

        lane = lax.iota(jnp.int32, _LANES)

        nc_per_p = npw // _LANES

        def stage_a(ci, _):
            p = ci // nc_per_p
            nc = ci - p * nc_per_p
            sn = pl.ds(nc * _LANES, _LANES)
            x = (xb[p, sn] - ox) / sx
            y = (yb[p, sn] - oy) / sy
            xi = x.astype(jnp.int32)
            yi = y.astype(jnp.int32)
            xi = jnp.where(xi.astype(jnp.float32) > x, xi - 1, xi)
            yi = jnp.where(yi.astype(jnp.float32) > y, yi - 1, yi)
            x0 = jnp.clip(xi, 0, W - 1)
            y0 = jnp.clip(yi, 0, H - 1)
            x1 = jnp.minimum(x0 + 1, W - 1)
            y1 = jnp.minimum(y0 + 1, H - 1)
            wx0 = x - x0.astype(jnp.float32)
            wx1 = x1.astype(jnp.float32) - x
            wy0 = y - y0.astype(jnp.float32)
            wy1 = y1.astype(jnp.float32) - y
            s = pl.ds(ci * _LANES, _LANES)
            wab[s] = wx1 * wy1
            wbb[s] = wx1 * wy0
            wcb[s] = wx0 * wy1
            wdb[s] = wx0 * wy0
            ra = (y0 * W + x0) * (CT * B) + b
            rb = (y1 * W + x0) * (CT * B) + b
            rc = (y0 * W + x1) * (CT * B) + b
            rd = (y1 * W + x1) * (CT * B) + b
            gidx[ci, pl.ds(0 * _LANES, _LANES)] = ra
            gidx[ci, pl.ds(1 * _LANES, _LANES)] = ra + B
            gidx[ci, pl.ds(2 * _LANES, _LANES)] = rb
            gidx[ci, pl.ds(3 * _LANES, _LANES)] = rb + B
            gidx[ci, pl.ds(4 * _LANES, _LANES)] = rc
            gidx[ci, pl.ds(5 * _LANES, _LANES)] = rc + B
            gidx[ci, pl.ds(6 * _LANES, _LANES)] = rd
            gidx[ci, pl.ds(7 * _LANES, _LANES)] = rd + B
            n = n0 + nc * _LANES + lane
            o0 = ((b * TN + lax.shift_right_logical(n, 3)) * TPC
                  + p * CT) * 8 + (n & 7)
            oidx[ci, pl.ds(0, _LANES)] = o0
            oidx[ci, pl.ds(_LANES, _LANES)] = o0 + 8
            return 0

        lax.fori_loop(0, nchunk, stage_a, 0)

        def combine(ci, gbuf, obuf):
            s = pl.ds(ci * _LANES, _LANES)
            wa = wab[s]
            wb = wbb[s]
            wc = wcb[s]
            wd = wdb[s]

            @plsc.parallel_loop(0, _LANES, step=1, unroll=2)
            def point(i):
                bi = jnp.full((_LANES,), i, jnp.int32)
                wai = wa.at[bi].get(mode="promise_in_bounds")
                wbi = wb.at[bi].get(mode="promise_in_bounds")
                wci = wc.at[bi].get(mode="promise_in_bounds")
                wdi = wd.at[bi].get(mode="promise_in_bounds")
                for t in range(CT):
                    r = t * _LANES + i
                    for v in range(_RL // _LANES):
                        cs = pl.ds(v * _LANES, _LANES)
                        acc = (gbuf[0 * CT * _LANES + r, cs] * wai
                               + gbuf[1 * CT * _LANES + r, cs] * wbi
                               + gbuf[2 * CT * _LANES + r, cs] * wci
                               + gbuf[3 * CT * _LANES + r, cs] * wdi)
                        obuf[r, cs] = acc

        last = nchunk - 1
        pltpu.async_copy(rows_hbm.at[gidx.at[0]], gb0, gs0)
        pltpu.async_copy(rows_hbm.at[gidx.at[1]], gb1, gs1)

        def pair(hi, _):
            for par, gbuf, gsem, obuf, osem in (
                    (0, gb0, gs0, ob0, os0), (1, gb1, gs1, ob1, os1)):
                ci = hi * 2 + par
                pltpu.make_async_copy(rows_hbm.at[gidx.at[ci]], gbuf,
                                      gsem).wait()

                @pl.when(hi > 0)
                def _wait_prev_scatter(obuf=obuf, osem=osem, ci=ci):
                    pltpu.make_async_copy(obuf, out_hbm.at[oidx.at[ci - 2]],
                                          osem).wait()

                combine(ci, gbuf, obuf)
                pltpu.async_copy(obuf, out_hbm.at[oidx.at[ci]], osem)
                nxt = jnp.minimum(ci + 2, last)
                pltpu.async_copy(rows_hbm.at[gidx.at[nxt]], gbuf, gsem)
            return 0

        lax.fori_loop(0, nchunk // 2, pair, 0)

        pltpu.make_async_copy(rows_hbm.at[gidx.at[last]], gb0, gs0).wait()
        pltpu.make_async_copy(rows_hbm.at[gidx.at[last]], gb1, gs1).wait()
        pltpu.make_async_copy(ob0, out_hbm.at[oidx.at[last - 1]], os0).wait()
        pltpu.make_async_copy(ob1, out_hbm.at[oidx.at[last]], os1).wait()

    return bev_kernel


def kernel(bev_feature, batch_centers, num_point):
    if isinstance(num_point, tuple):
        num_point = num_point[0] * num_point[1]
    B, C, H, W = bev_feature.shape
    _, N, P, _ = batch_centers.shape
    CT = C // _RL
    TN = N // 8
    TPC = P * C // _RL
    bev_rows = jnp.transpose(
        bev_feature.reshape(B, CT, _RL, H, W), (3, 4, 1, 0, 2)
    ).reshape(H * W * CT * B, _RL)
    bc_view = jnp.transpose(
        batch_centers.reshape(B, N // _RL, _RL, P, 2), (0, 3, 1, 4, 2))
    out_rows = _make_sc_kernel(B, C, H, W, N, P)(bev_rows, bc_view)
    return jnp.transpose(
        out_rows.reshape(B, TN, TPC, 8, _RL), (0, 1, 3, 2, 4)
    ).reshape(B, N, P * C)


# speedup vs baseline: 10.8828x; 1.0097x over previous
import functools

import jax
import jax.numpy as jnp
from jax import lax
from jax.experimental import pallas as pl
from jax.experimental.pallas import tpu as pltpu
from jax.experimental.pallas import tpu_sc as plsc

_PC_START = (-54.0, -54.0)
_VOXEL_SIZE = (0.075, 0.075)
_OUT_STRIDE = 8

_LANES = 16
_RL = 128


@functools.cache
def _make_sc_kernel(B, C, H, W, N, P):
    info = plsc.get_sparse_core_info()
    NC, NS = info.num_cores, info.num_subcores
    NW = NC * NS
    NPTS = N * P
    CT = C // _RL
    TN = N // 8
    TPC = P * C // _RL
    ppw = B * NPTS // NW
    nchunk = ppw // _LANES
    wpb = NW // B
    npw = N // wpb
    assert CT * _RL == C and TN * 8 == N and wpb * B == NW
    assert ppw * NW == B * NPTS and nchunk % 2 == 0
    assert npw == _RL and nchunk == P * (npw // _LANES)

    sx = float(_VOXEL_SIZE[0] * _OUT_STRIDE)
    sy = float(_VOXEL_SIZE[1] * _OUT_STRIDE)
    ox = float(_PC_START[0])
    oy = float(_PC_START[1])

    mesh = plsc.VectorSubcoreMesh(core_axis_name="c", subcore_axis_name="s")

    @functools.partial(
        pl.kernel,
        out_type=jax.ShapeDtypeStruct((B * TN * TPC * 8, _RL), jnp.float32),
        mesh=mesh,
        compiler_params=pltpu.CompilerParams(use_tc_tiling_on_sc=False,
                                             needs_layout_passes=False),
        scratch_types=[
            pltpu.VMEM((P, npw), jnp.float32),
            pltpu.VMEM((P, npw), jnp.float32),
            pltpu.VMEM((ppw,), jnp.float32),
            pltpu.VMEM((ppw,), jnp.float32),
            pltpu.VMEM((ppw,), jnp.float32),
            pltpu.VMEM((ppw,), jnp.float32),
            pltpu.VMEM((nchunk, 8 * _LANES), jnp.int32),
            pltpu.VMEM((nchunk, 2 * _LANES), jnp.int32),
            pltpu.VMEM((8 * _LANES, _RL), jnp.float32),
            pltpu.VMEM((8 * _LANES, _RL), jnp.float32),
            pltpu.VMEM((2 * _LANES, _RL), jnp.float32),
            pltpu.VMEM((2 * _LANES, _RL), jnp.float32),
            pltpu.SemaphoreType.DMA,
            pltpu.SemaphoreType.DMA,
            pltpu.SemaphoreType.DMA,
            pltpu.SemaphoreType.DMA,
        ],
    )
    def bev_kernel(rows_hbm, bc_hbm, out_hbm,
                   xb, yb, wab, wbb, wcb, wdb, gidx, oidx,
                   gb0, gb1, ob0, ob1, gs0, gs1, os0, os1):
        wid = lax.axis_index("s") * NC + lax.axis_index("c")
        b = wid // wpb
        n0 = (wid % wpb) * npw

        nb = wid % wpb
        pltpu.sync_copy(bc_hbm.at[b, :, nb, 0, :], xb)
        pltpu.sync_copy(bc_hbm.at[b, :, nb, 1, :], yb)

        lane = lax.iota(jnp.int32, _LANES)

        nc_per_p = npw // _LANES

        def stage_a(ci, _):
            p = ci // nc_per_p
            nc = ci - p * nc_per_p
            sn = pl.ds(nc * _LANES, _LANES)
            x = (xb[p, sn] - ox) / sx
            y = (yb[p, sn] - oy) / sy
            xi = x.astype(jnp.int32)
            yi = y.astype(jnp.int32)
            xi = jnp.where(xi.astype(jnp.float32) > x, xi - 1, xi)
            yi = jnp.where(yi.astype(jnp.float32) > y, yi - 1, yi)
            x0 = jnp.clip(xi, 0, W - 1)
            y0 = jnp.clip(yi, 0, H - 1)
            x1 = jnp.minimum(x0 + 1, W - 1)
            y1 = jnp.minimum(y0 + 1, H - 1)
            wx0 = x - x0.astype(jnp.float32)
            wx1 = x1.astype(jnp.float32) - x
            wy0 = y - y0.astype(jnp.float32)
            wy1 = y1.astype(jnp.float32) - y
            s = pl.ds(ci * _LANES, _LANES)
            wab[s] = wx1 * wy1
            wbb[s] = wx1 * wy0
            wcb[s] = wx0 * wy1
            wdb[s] = wx0 * wy0
            ra = (y0 * W + x0) * (CT * B) + b
            rb = (y1 * W + x0) * (CT * B) + b
            rc = (y0 * W + x1) * (CT * B) + b
            rd = (y1 * W + x1) * (CT * B) + b
            gidx[ci, pl.ds(0 * _LANES, _LANES)] = ra
            gidx[ci, pl.ds(1 * _LANES, _LANES)] = ra + B
            gidx[ci, pl.ds(2 * _LANES, _LANES)] = rb
            gidx[ci, pl.ds(3 * _LANES, _LANES)] = rb + B
            gidx[ci, pl.ds(4 * _LANES, _LANES)] = rc
            gidx[ci, pl.ds(5 * _LANES, _LANES)] = rc + B
            gidx[ci, pl.ds(6 * _LANES, _LANES)] = rd
            gidx[ci, pl.ds(7 * _LANES, _LANES)] = rd + B
            n = n0 + nc * _LANES + lane
            o0 = ((b * TN + lax.shift_right_logical(n, 3)) * TPC
                  + p * CT) * 8 + (n & 7)
            oidx[ci, pl.ds(0, _LANES)] = o0
            oidx[ci, pl.ds(_LANES, _LANES)] = o0 + 8
            return 0

        lax.fori_loop(0, nchunk, stage_a, 0)

        def combine(ci, gbuf, obuf):
            s = pl.ds(ci * _LANES, _LANES)
            wa = wab[s]
            wb = wbb[s]
            wc = wcb[s]
            wd = wdb[s]

            @plsc.parallel_loop(0, _LANES, step=1, unroll=1)
            def point(i):
                bi = jnp.full((_LANES,), i, jnp.int32)
                wai = wa.at[bi].get(mode="promise_in_bounds")
                wbi = wb.at[bi].get(mode="promise_in_bounds")
                wci = wc.at[bi].get(mode="promise_in_bounds")
                wdi = wd.at[bi].get(mode="promise_in_bounds")
                for t in range(CT):
                    r = t * _LANES + i
                    for v in range(_RL // _LANES):
                        cs = pl.ds(v * _LANES, _LANES)
                        acc = (gbuf[0 * CT * _LANES + r, cs] * wai
                               + gbuf[1 * CT * _LANES + r, cs] * wbi
                               + gbuf[2 * CT * _LANES + r, cs] * wci
                               + gbuf[3 * CT * _LANES + r, cs] * wdi)
                        obuf[r, cs] = acc

        last = nchunk - 1
        pltpu.async_copy(rows_hbm.at[gidx.at[0]], gb0, gs0)
        pltpu.async_copy(rows_hbm.at[gidx.at[1]], gb1, gs1)

        def pair(hi, _):
            for par, gbuf, gsem, obuf, osem in (
                    (0, gb0, gs0, ob0, os0), (1, gb1, gs1, ob1, os1)):
                ci = hi * 2 + par
                pltpu.make_async_copy(rows_hbm.at[gidx.at[ci]], gbuf,
                                      gsem).wait()

                @pl.when(hi > 0)
                def _wait_prev_scatter(obuf=obuf, osem=osem, ci=ci):
                    pltpu.make_async_copy(obuf, out_hbm.at[oidx.at[ci - 2]],
                                          osem).wait()

                combine(ci, gbuf, obuf)
                pltpu.async_copy(obuf, out_hbm.at[oidx.at[ci]], osem)
                nxt = jnp.minimum(ci + 2, last)
                pltpu.async_copy(rows_hbm.at[gidx.at[nxt]], gbuf, gsem)
            return 0

        lax.fori_loop(0, nchunk // 2, pair, 0)

        pltpu.make_async_copy(rows_hbm.at[gidx.at[last]], gb0, gs0).wait()
        pltpu.make_async_copy(rows_hbm.at[gidx.at[last]], gb1, gs1).wait()
        pltpu.make_async_copy(ob0, out_hbm.at[oidx.at[last - 1]], os0).wait()
        pltpu.make_async_copy(ob1, out_hbm.at[oidx.at[last]], os1).wait()

    return bev_kernel


def kernel(bev_feature, batch_centers, num_point):
    if isinstance(num_point, tuple):
        num_point = num_point[0] * num_point[1]
    B, C, H, W = bev_feature.shape
    _, N, P, _ = batch_centers.shape
    CT = C // _RL
    TN = N // 8
    TPC = P * C // _RL
    bev_rows = jnp.transpose(
        bev_feature.reshape(B, CT, _RL, H, W), (3, 4, 1, 0, 2)
    ).reshape(H * W * CT * B, _RL)
    bc_view = jnp.transpose(
        batch_centers.reshape(B, N // _RL, _RL, P, 2), (0, 3, 1, 4, 2))
    out_rows = _make_sc_kernel(B, C, H, W, N, P)(bev_rows, bc_view)
    return jnp.transpose(
        out_rows.reshape(B, TN, TPC, 8, _RL), (0, 1, 3, 2, 4)
    ).reshape(B, N, P * C)


# speedup vs baseline: 11.0452x; 1.0149x over previous
import functools

import jax
import jax.numpy as jnp
from jax import lax
from jax.experimental import pallas as pl
from jax.experimental.pallas import tpu as pltpu
from jax.experimental.pallas import tpu_sc as plsc

_PC_START = (-54.0, -54.0)
_VOXEL_SIZE = (0.075, 0.075)
_OUT_STRIDE = 8

_LANES = 16
_RL = 128


@functools.cache
def _make_sc_kernel(B, C, H, W, N, P):
    info = plsc.get_sparse_core_info()
    NC, NS = info.num_cores, info.num_subcores
    NW = NC * NS
    NPTS = N * P
    CT = C // _RL
    TN = N // 8
    TPC = P * C // _RL
    ppw = B * NPTS // NW
    nchunk = ppw // _LANES
    wpb = NW // B
    npw = N // wpb
    assert CT * _RL == C and TN * 8 == N and wpb * B == NW
    assert ppw * NW == B * NPTS and nchunk % 2 == 0
    assert npw == _RL and nchunk == P * (npw // _LANES)

    sx = float(_VOXEL_SIZE[0] * _OUT_STRIDE)
    sy = float(_VOXEL_SIZE[1] * _OUT_STRIDE)
    ox = float(_PC_START[0])
    oy = float(_PC_START[1])

    mesh = plsc.VectorSubcoreMesh(core_axis_name="c", subcore_axis_name="s")

    @functools.partial(
        pl.kernel,
        out_type=jax.ShapeDtypeStruct((B * TN * TPC * 8, _RL), jnp.float32),
        mesh=mesh,
        compiler_params=pltpu.CompilerParams(use_tc_tiling_on_sc=False,
                                             needs_layout_passes=False),
        scratch_types=[
            pltpu.VMEM((P, 2, npw), jnp.float32),
            pltpu.VMEM((ppw,), jnp.float32),
            pltpu.VMEM((ppw,), jnp.float32),
            pltpu.VMEM((ppw,), jnp.float32),
            pltpu.VMEM((ppw,), jnp.float32),
            pltpu.VMEM((nchunk, 8 * _LANES), jnp.int32),
            pltpu.VMEM((nchunk, 2 * _LANES), jnp.int32),
            pltpu.VMEM((8 * _LANES, _RL), jnp.float32),
            pltpu.VMEM((8 * _LANES, _RL), jnp.float32),
            pltpu.VMEM((2 * _LANES, _RL), jnp.float32),
            pltpu.VMEM((2 * _LANES, _RL), jnp.float32),
            pltpu.SemaphoreType.DMA,
            pltpu.SemaphoreType.DMA,
            pltpu.SemaphoreType.DMA,
            pltpu.SemaphoreType.DMA,
        ],
    )
    def bev_kernel(rows_hbm, bc_hbm, out_hbm,
                   bcb, wab, wbb, wcb, wdb, gidx, oidx,
                   gb0, gb1, ob0, ob1, gs0, gs1, os0, os1):
        wid = lax.axis_index("s") * NC + lax.axis_index("c")
        b = wid // wpb
        nb = wid % wpb
        n0 = nb * npw

        pltpu.sync_copy(bc_hbm.at[b, :, nb], bcb)

        lane = lax.iota(jnp.int32, _LANES)

        nc_per_p = npw // _LANES

        def stage_a(ci, _):
            p = ci // nc_per_p
            nc = ci - p * nc_per_p
            sn = pl.ds(nc * _LANES, _LANES)
            x = (bcb[p, 0, sn] - ox) / sx
            y = (bcb[p, 1, sn] - oy) / sy
            xi = x.astype(jnp.int32)
            yi = y.astype(jnp.int32)
            xi = jnp.where(xi.astype(jnp.float32) > x, xi - 1, xi)
            yi = jnp.where(yi.astype(jnp.float32) > y, yi - 1, yi)
            x0 = jnp.clip(xi, 0, W - 1)
            y0 = jnp.clip(yi, 0, H - 1)
            x1 = jnp.minimum(x0 + 1, W - 1)
            y1 = jnp.minimum(y0 + 1, H - 1)
            wx0 = x - x0.astype(jnp.float32)
            wx1 = x1.astype(jnp.float32) - x
            wy0 = y - y0.astype(jnp.float32)
            wy1 = y1.astype(jnp.float32) - y
            s = pl.ds(ci * _LANES, _LANES)
            wab[s] = wx1 * wy1
            wbb[s] = wx1 * wy0
            wcb[s] = wx0 * wy1
            wdb[s] = wx0 * wy0
            ra = (y0 * W + x0) * (CT * B) + b
            rb = (y1 * W + x0) * (CT * B) + b
            rc = (y0 * W + x1) * (CT * B) + b
            rd = (y1 * W + x1) * (CT * B) + b
            gidx[ci, pl.ds(0 * _LANES, _LANES)] = ra
            gidx[ci, pl.ds(1 * _LANES, _LANES)] = ra + B
            gidx[ci, pl.ds(2 * _LANES, _LANES)] = rb
            gidx[ci, pl.ds(3 * _LANES, _LANES)] = rb + B
            gidx[ci, pl.ds(4 * _LANES, _LANES)] = rc
            gidx[ci, pl.ds(5 * _LANES, _LANES)] = rc + B
            gidx[ci, pl.ds(6 * _LANES, _LANES)] = rd
            gidx[ci, pl.ds(7 * _LANES, _LANES)] = rd + B
            n = n0 + nc * _LANES + lane
            o0 = ((b * TN + lax.shift_right_logical(n, 3)) * TPC
                  + p * CT) * 8 + (n & 7)
            oidx[ci, pl.ds(0, _LANES)] = o0
            oidx[ci, pl.ds(_LANES, _LANES)] = o0 + 8

        def combine(ci, gbuf, obuf):
            s = pl.ds(ci * _LANES, _LANES)
            wa = wab[s]
            wb = wbb[s]
            wc = wcb[s]
            wd = wdb[s]

            @plsc.parallel_loop(0, _LANES, step=1, unroll=1)
            def point(i):
                bi = jnp.full((_LANES,), i, jnp.int32)
                wai = wa.at[bi].get(mode="promise_in_bounds")
                wbi = wb.at[bi].get(mode="promise_in_bounds")
                wci = wc.at[bi].get(mode="promise_in_bounds")
                wdi = wd.at[bi].get(mode="promise_in_bounds")
                for t in range(CT):
                    r = t * _LANES + i
                    for v in range(_RL // _LANES):
                        cs = pl.ds(v * _LANES, _LANES)
                        acc = (gbuf[0 * CT * _LANES + r, cs] * wai
                               + gbuf[1 * CT * _LANES + r, cs] * wbi
                               + gbuf[2 * CT * _LANES + r, cs] * wci
                               + gbuf[3 * CT * _LANES + r, cs] * wdi)
                        obuf[r, cs] = acc

        last = nchunk - 1
        stage_a(0, None)
        stage_a(1, None)
        pltpu.async_copy(rows_hbm.at[gidx.at[0]], gb0, gs0)
        pltpu.async_copy(rows_hbm.at[gidx.at[1]], gb1, gs1)

        def pair(hi, _):
            for par, gbuf, gsem, obuf, osem in (
                    (0, gb0, gs0, ob0, os0), (1, gb1, gs1, ob1, os1)):
                ci = hi * 2 + par
                pltpu.make_async_copy(rows_hbm.at[gidx.at[ci]], gbuf,
                                      gsem).wait()

                @pl.when(hi > 0)
                def _wait_prev_scatter(obuf=obuf, osem=osem, ci=ci):
                    pltpu.make_async_copy(obuf, out_hbm.at[oidx.at[ci - 2]],
                                          osem).wait()

                combine(ci, gbuf, obuf)
                pltpu.async_copy(obuf, out_hbm.at[oidx.at[ci]], osem)

                @pl.when(ci + 2 <= last)
                def _prep_next(ci=ci):
                    stage_a(ci + 2, None)

                nxt = jnp.minimum(ci + 2, last)
                pltpu.async_copy(rows_hbm.at[gidx.at[nxt]], gbuf, gsem)
            return 0

        lax.fori_loop(0, nchunk // 2, pair, 0)

        pltpu.make_async_copy(rows_hbm.at[gidx.at[last]], gb0, gs0).wait()
        pltpu.make_async_copy(rows_hbm.at[gidx.at[last]], gb1, gs1).wait()
        pltpu.make_async_copy(ob0, out_hbm.at[oidx.at[last - 1]], os0).wait()
        pltpu.make_async_copy(ob1, out_hbm.at[oidx.at[last]], os1).wait()

    return bev_kernel


def kernel(bev_feature, batch_centers, num_point):
    if isinstance(num_point, tuple):
        num_point = num_point[0] * num_point[1]
    B, C, H, W = bev_feature.shape
    _, N, P, _ = batch_centers.shape
    CT = C // _RL
    TN = N // 8
    TPC = P * C // _RL
    bev_rows = jnp.transpose(
        bev_feature.reshape(B, CT, _RL, H, W), (3, 4, 1, 0, 2)
    ).reshape(H * W * CT * B, _RL)
    bc_view = jnp.transpose(
        batch_centers.reshape(B, N // _RL, _RL, P, 2), (0, 3, 1, 4, 2))
    out_rows = _make_sc_kernel(B, C, H, W, N, P)(bev_rows, bc_view)
    return jnp.transpose(
        out_rows.reshape(B, TN, TPC, 8, _RL), (0, 1, 3, 2, 4)
    ).reshape(B, N, P * C)
